# Initial kernel scaffold; baseline (speedup 1.0000x reference)
import jax
import jax.numpy as jnp
from jax.experimental import pallas as pl


def kernel(token_ids, weight):
    raise NotImplementedError("write your pallas kernel here")


# speedup vs baseline: 1.1035x; 1.1035x over previous
import functools

import jax
import jax.numpy as jnp
from jax import lax
from jax.experimental import pallas as pl
from jax.experimental.pallas import tpu as pltpu
from jax.experimental.pallas import tpu_sc as plsc

NUM_EMB = 1000000
DIM = 32
B = 16384 * 50
NC, NS = 2, 16
NW = NC * NS
B_PER_W = B // NW
CHUNK = 1600
N_CHUNKS = B_PER_W // CHUNK


@functools.partial(
    pl.kernel,
    mesh=plsc.VectorSubcoreMesh(core_axis_name="c", subcore_axis_name="s"),
    out_type=jax.ShapeDtypeStruct((B, DIM), jnp.float32),
    compiler_params=pltpu.CompilerParams(use_tc_tiling_on_sc=False),
    scratch_types=[
        pltpu.VMEM((CHUNK,), jnp.int32),
        pltpu.VMEM((CHUNK, DIM), jnp.float32),
        pltpu.SemaphoreType.DMA,
    ],
)
def _gather_sc(table_hbm, idx_hbm, out_hbm, idx_v, rows_v, sem):
    wid = lax.axis_index("s") * NC + lax.axis_index("c")
    base = wid * B_PER_W

    def body(c, carry):
        off = base + c * CHUNK
        pltpu.sync_copy(idx_hbm.at[pl.ds(off, CHUNK)], idx_v)
        pltpu.async_copy(table_hbm.at[idx_v], rows_v, sem).wait()
        pltpu.sync_copy(rows_v, out_hbm.at[pl.ds(off, CHUNK)])
        return carry

    lax.fori_loop(0, N_CHUNKS, body, 0)


def kernel(token_ids, weight):
    idx = token_ids.reshape(-1).astype(jnp.int32)
    out = _gather_sc(weight, idx)
    return out.reshape(token_ids.shape[0], token_ids.shape[1], DIM)


# speedup vs baseline: 1.1097x; 1.0057x over previous
import functools

import jax
import jax.numpy as jnp
from jax import lax
from jax.experimental import pallas as pl
from jax.experimental.pallas import tpu as pltpu
from jax.experimental.pallas import tpu_sc as plsc

NUM_EMB = 1000000
DIM = 32
B = 16384 * 50
NC, NS = 2, 16
NW = NC * NS
B_PER_W = B // NW
CHUNK = 1600
N_CHUNKS = B_PER_W // CHUNK


@functools.partial(
    pl.kernel,
    mesh=plsc.VectorSubcoreMesh(core_axis_name="c", subcore_axis_name="s"),
    out_type=jax.ShapeDtypeStruct((B, DIM), jnp.float32),
    compiler_params=pltpu.CompilerParams(use_tc_tiling_on_sc=False),
    scratch_types=[
        pltpu.VMEM((N_CHUNKS, CHUNK), jnp.int32),
        pltpu.VMEM((CHUNK, DIM), jnp.float32),
        pltpu.VMEM((CHUNK, DIM), jnp.float32),
        pltpu.SemaphoreType.DMA,
        pltpu.SemaphoreType.DMA,
        pltpu.SemaphoreType.DMA,
    ],
)
def _gather_sc(table_hbm, idx_hbm, out_hbm, idx_v, rows0, rows1, sem_g, sem_w0, sem_w1):
    wid = lax.axis_index("s") * NC + lax.axis_index("c")
    base = wid * B_PER_W
    pltpu.sync_copy(idx_hbm.at[wid], idx_v)

    bufs = (rows0, rows1)
    wsems = (sem_w0, sem_w1)

    def gather(c):
        return pltpu.async_copy(table_hbm.at[idx_v.at[c]], bufs[c % 2], sem_g)

    writes = [None] * N_CHUNKS
    pending = gather(0)
    for c in range(N_CHUNKS):
        pending.wait()
        if c >= 1:
            writes[c - 1].wait()
        writes[c] = pltpu.async_copy(
            bufs[c % 2], out_hbm.at[pl.ds(base + c * CHUNK, CHUNK)], wsems[c % 2]
        )
        if c + 1 < N_CHUNKS:
            pending = gather(c + 1)
    writes[N_CHUNKS - 1].wait()


def kernel(token_ids, weight):
    idx = token_ids.reshape(NW, N_CHUNKS, CHUNK).astype(jnp.int32)
    out = _gather_sc(weight, idx)
    return out.reshape(token_ids.shape[0], token_ids.shape[1], DIM)


# speedup vs baseline: 1.7888x; 1.6119x over previous
import functools

import jax
import jax.numpy as jnp
from jax import lax
from jax.experimental import pallas as pl
from jax.experimental.pallas import tpu as pltpu
from jax.experimental.pallas import tpu_sc as plsc

NUM_EMB = 1000000
DIM = 32
ROWS, COLS = 16384, 50
B = ROWS * COLS
NC, NS = 2, 16
NW = NC * NS
R_PER_W = ROWS // NW
T_PER_C = 32
CHUNK = T_PER_C * COLS
N_CHUNKS = R_PER_W // T_PER_C


@functools.partial(
    pl.kernel,
    mesh=plsc.VectorSubcoreMesh(core_axis_name="c", subcore_axis_name="s"),
    out_type=jax.ShapeDtypeStruct((ROWS, COLS, DIM), jnp.float32),
    compiler_params=pltpu.CompilerParams(use_tc_tiling_on_sc=False),
    scratch_types=[
        pltpu.VMEM((N_CHUNKS, CHUNK), jnp.int32),
        pltpu.VMEM((CHUNK, DIM), jnp.float32),
        pltpu.VMEM((CHUNK, DIM), jnp.float32),
        pltpu.SemaphoreType.DMA,
        pltpu.SemaphoreType.DMA,
        pltpu.SemaphoreType.DMA,
    ],
)
def _gather_sc(table_hbm, idx_hbm, out_hbm, idx_v, rows0, rows1, sem_g, sem_w0, sem_w1):
    wid = lax.axis_index("s") * NC + lax.axis_index("c")
    base = wid * R_PER_W * COLS
    for c in range(N_CHUNKS):
        pltpu.sync_copy(idx_hbm.at[pl.ds(base + c * CHUNK, CHUNK)], idx_v.at[c])

    bufs = (rows0, rows1)
    wsems = (sem_w0, sem_w1)

    def gather(c):
        return pltpu.async_copy(table_hbm.at[idx_v.at[c]], bufs[c % 2], sem_g)

    def issue_writes(c):
        t0 = wid * R_PER_W + c * T_PER_C
        buf = bufs[c % 2]
        return [
            pltpu.async_copy(
                buf.at[pl.ds(j * COLS, COLS), :], out_hbm.at[t0 + j], wsems[c % 2]
            )
            for j in range(T_PER_C)
        ]

    writes = [None] * N_CHUNKS
    pending = gather(0)
    for c in range(N_CHUNKS):
        pending.wait()
        if c >= 1:
            for w in writes[c - 1]:
                w.wait()
        if c + 1 < N_CHUNKS:
            pending = gather(c + 1)
        writes[c] = issue_writes(c)
    for w in writes[N_CHUNKS - 1]:
        w.wait()


def kernel(token_ids, weight):
    idx = token_ids.reshape(-1).astype(jnp.int32)
    return _gather_sc(weight, idx)


# speedup vs baseline: 2.5077x; 1.4019x over previous
import functools

import jax
import jax.numpy as jnp
from jax import lax
from jax.experimental import pallas as pl
from jax.experimental.pallas import tpu as pltpu
from jax.experimental.pallas import tpu_sc as plsc

NUM_EMB = 1000000
DIM = 32
ROWS, COLS = 16384, 50
PAD_COLS, PAD_DIM = 56, 128
B = ROWS * COLS
NC, NS = 2, 16
NW = NC * NS
R_PER_W = ROWS // NW
T_PER_C = 32
CHUNK = T_PER_C * COLS
N_CHUNKS = R_PER_W // T_PER_C


@functools.partial(
    pl.kernel,
    mesh=plsc.VectorSubcoreMesh(core_axis_name="c", subcore_axis_name="s"),
    out_type=jax.ShapeDtypeStruct((ROWS, PAD_COLS, PAD_DIM), jnp.float32),
    compiler_params=pltpu.CompilerParams(use_tc_tiling_on_sc=False),
    scratch_types=[
        pltpu.VMEM((N_CHUNKS, CHUNK), jnp.int32),
        pltpu.VMEM((CHUNK, DIM), jnp.float32),
        pltpu.VMEM((CHUNK, DIM), jnp.float32),
        pltpu.SemaphoreType.DMA,
        pltpu.SemaphoreType.DMA,
        pltpu.SemaphoreType.DMA,
    ],
)
def _gather_sc(table_hbm, idx_hbm, out_hbm, idx_v, rows0, rows1, sem_g, sem_w0, sem_w1):
    wid = lax.axis_index("s") * NC + lax.axis_index("c")
    base = wid * R_PER_W * COLS
    for c in range(N_CHUNKS):
        pltpu.sync_copy(idx_hbm.at[pl.ds(base + c * CHUNK, CHUNK)], idx_v.at[c])

    bufs = (rows0, rows1)
    wsems = (sem_w0, sem_w1)

    def gather(c):
        return pltpu.async_copy(table_hbm.at[idx_v.at[c]], bufs[c % 2], sem_g)

    def issue_writes(c):
        t0 = wid * R_PER_W + c * T_PER_C
        buf = bufs[c % 2]
        return [
            pltpu.async_copy(
                buf.at[pl.ds(j * COLS, COLS), :],
                out_hbm.at[t0 + j, pl.ds(0, COLS), pl.ds(0, DIM)],
                wsems[c % 2],
            )
            for j in range(T_PER_C)
        ]

    writes = [None] * N_CHUNKS
    pending = gather(0)
    for c in range(N_CHUNKS):
        pending.wait()
        if c >= 1:
            for w in writes[c - 1]:
                w.wait()
        if c + 1 < N_CHUNKS:
            pending = gather(c + 1)
        writes[c] = issue_writes(c)
    for w in writes[N_CHUNKS - 1]:
        w.wait()


def kernel(token_ids, weight):
    idx = jnp.minimum(token_ids.reshape(-1), NUM_EMB - 1).astype(jnp.int32)
    out = _gather_sc(weight, idx)
    return out[:, :COLS, :DIM]
